# Initial kernel scaffold; baseline (speedup 1.0000x reference)
import jax
import jax.numpy as jnp
from jax.experimental import pallas as pl


def kernel(x, params):
    raise NotImplementedError("write your pallas kernel here")


# speedup vs baseline: 1.5718x; 1.5718x over previous
import functools

import jax
import jax.numpy as jnp
from jax.experimental import pallas as pl

_EPS = 1e-6
_BN_EPS = 1e-5
_NKNN = 10
_KPAD = 16
_NCLASS = 40


def _knn_kernel_body(x_ref, idx_ref, *, n, k):
    X = x_ref[0]
    G = jax.lax.dot_general(
        X, X, (((0,), (0,)), ((), ())), preferred_element_type=jnp.float32
    )
    xx = jnp.sum(X * X, axis=0)
    inner = -2.0 * G
    pairwise = -xx[:, None] - inner - xx[None, :]
    t = -pairwise
    iota = jax.lax.broadcasted_iota(jnp.int32, (n, n), 1)
    neg = jnp.float32(-jnp.inf)
    for j in range(k):
        m = jnp.max(t, axis=1, keepdims=True)
        cand = jnp.where(t == m, iota, n)
        am = jnp.min(cand, axis=1)
        idx_ref[0, j, :] = am
        t = jnp.where(iota == am[:, None], neg, t)
    for j in range(k, _KPAD):
        idx_ref[0, j, :] = jnp.zeros((n,), jnp.int32)


def _knn_idx(xf, k):
    B, D, N = xf.shape
    Dp = ((D + 7) // 8) * 8
    if Dp != D:
        xf = jnp.pad(xf, ((0, 0), (0, Dp - D), (0, 0)))
    out = pl.pallas_call(
        functools.partial(_knn_kernel_body, n=N, k=k),
        grid=(B,),
        in_specs=[pl.BlockSpec((1, Dp, N), lambda b: (b, 0, 0))],
        out_specs=pl.BlockSpec((1, _KPAD, N), lambda b: (b, 0, 0)),
        out_shape=jax.ShapeDtypeStruct((B, _KPAD, N), jnp.int32),
    )(xf)
    return jnp.transpose(out[:, :k, :], (0, 2, 1))


def _graph_feature(x, k):
    B, C, _, N = x.shape
    xf = x.reshape(B, C * 3, N)
    idx = _knn_idx(xf, k)
    xt = jnp.transpose(xf, (0, 2, 1))
    feature = jax.vmap(lambda xb, ib: jnp.take(xb, ib, axis=0))(xt, idx)
    feature = feature.reshape(B, N, k, C, 3)
    xc = jnp.broadcast_to(xt.reshape(B, N, 1, C, 3), (B, N, k, C, 3))
    out = jnp.concatenate([feature - xc, xc], axis=3)
    return jnp.transpose(out, (0, 3, 4, 1, 2))


def _vn_lin_lrelu(x, Wf, Wd, gamma, beta, negative_slope=0.2):
    p = jnp.einsum('oi,bi...->bo...', Wf, x)
    norm = jnp.sqrt(jnp.sum(p * p, axis=2)) + _EPS
    axes = (0,) + tuple(range(2, norm.ndim))
    mean = jnp.mean(norm, axis=axes, keepdims=True)
    var = jnp.var(norm, axis=axes, keepdims=True)
    shp = (1, -1) + (1,) * (norm.ndim - 2)
    norm_bn = (norm - mean) / jnp.sqrt(var + _BN_EPS) * gamma.reshape(shp) + beta.reshape(shp)
    p = p / norm[:, :, None] * norm_bn[:, :, None]
    d = jnp.einsum('oi,bi...->bo...', Wd, x)
    dot = jnp.sum(p * d, axis=2, keepdims=True)
    mask = (dot >= 0).astype(p.dtype)
    d2 = jnp.sum(d * d, axis=2, keepdims=True)
    return negative_slope * p + (1.0 - negative_slope) * (
        mask * p + (1.0 - mask) * (p - (dot / (d2 + _EPS)) * d)
    )


def _bn1d(v, g, b):
    m = jnp.mean(v, axis=0, keepdims=True)
    va = jnp.var(v, axis=0, keepdims=True)
    return (v - m) / jnp.sqrt(va + _BN_EPS) * g + b


def _lrelu(v):
    return jnp.where(v >= 0, v, 0.2 * v)


def kernel(x, params):
    p = params
    B, N = x.shape[0], x.shape[1]
    k = min(_NKNN, N - 1)
    h = jnp.transpose(x, (0, 2, 1))[:, None, :, :]
    h = _vn_lin_lrelu(_graph_feature(h, k), p['c1_Wf'], p['c1_Wd'], p['c1_g'], p['c1_b'])
    x1 = jnp.mean(h, axis=-1)
    h = _vn_lin_lrelu(_graph_feature(x1, k), p['c2_Wf'], p['c2_Wd'], p['c2_g'], p['c2_b'])
    x2 = jnp.mean(h, axis=-1)
    h = _vn_lin_lrelu(_graph_feature(x2, k), p['c3_Wf'], p['c3_Wd'], p['c3_g'], p['c3_b'])
    x3 = jnp.mean(h, axis=-1)
    h = _vn_lin_lrelu(_graph_feature(x3, k), p['c4_Wf'], p['c4_Wd'], p['c4_g'], p['c4_b'])
    x4 = jnp.mean(h, axis=-1)
    h = jnp.concatenate([x1, x2, x3, x4], axis=1)
    h = _vn_lin_lrelu(h, p['c5_Wf'], p['c5_Wd'], p['c5_g'], p['c5_b'])
    hm = jnp.broadcast_to(jnp.mean(h, axis=-1, keepdims=True), h.shape)
    h = jnp.concatenate([h, hm], axis=1)
    z = _vn_lin_lrelu(h, p['s1_Wf'], p['s1_Wd'], p['s1_g'], p['s1_b'])
    z = _vn_lin_lrelu(z, p['s2_Wf'], p['s2_Wd'], p['s2_g'], p['s2_b'])
    z = jnp.einsum('oi,bi...->bo...', p['s_Wlin'], z)
    z = jnp.swapaxes(z, 1, 2)
    hs = jnp.einsum('bijm,bjkm->bikm', h, z)
    hf = hs.reshape(B, -1, N)
    feat = jnp.concatenate([jnp.max(hf, axis=-1), jnp.mean(hf, axis=-1)], axis=1)
    feat = _lrelu(_bn1d(feat @ p['l1_W'].T + p['l1_b'], p['bn1_g'], p['bn1_b']))
    feat = _lrelu(_bn1d(feat @ p['l2_W'].T + p['l2_b'], p['bn2_g'], p['bn2_b']))
    return feat @ p['l3_W'].T + p['l3_b']
